# baseline (speedup 1.0000x reference)
import functools
import math

import jax
import jax.numpy as jnp
from jax import lax
from jax.experimental import pallas as pl
from jax.experimental.pallas import tpu as pltpu
from jax.experimental.pallas import tpu_sc as plsc

_B = 16
_N = 32768
_D = 1024
_TEMPERATURE = 1.0

_L = 16
_NTILES = 16
_CHUNK = _N // _NTILES
_NVEC = _CHUNK // _L

_NEG_INF = float("-inf")


def _tc_prep_body(std_ref, batch_ref, unc_ref, starts_ref):
    s = std_ref[...]
    ent = 0.5 * jnp.log((2.0 * math.pi * math.e) * jnp.square(s))
    unc_ref[...] = jnp.sum(ent, axis=1, keepdims=True)

    b2 = batch_ref[...]
    iota2 = lax.broadcasted_iota(jnp.int32, (_B, 1), 0)
    acc = jnp.zeros((_B, 1), jnp.int32)
    for b in range(_B):
        cnt = jnp.sum((b2 < b).astype(jnp.int32))
        acc = jnp.where(iota2 == b, cnt, acc)
    starts_ref[...] = acc


def _tc_prep(posterior_std, batch):
    unc, starts = pl.pallas_call(
        _tc_prep_body,
        out_shape=[
            jax.ShapeDtypeStruct((_B, 1), jnp.float32),
            jax.ShapeDtypeStruct((_B, 1), jnp.int32),
        ],
    )(posterior_std, batch.reshape(_B * _L, -1))
    return unc.reshape(_B), starts.reshape(_B)


def _sc_body(coh_hbm, batch_hbm, u_hbm, starts_hbm, prio_hbm, norm_hbm,
             coh_v, idx_v, s_v, e_v, n_v,
             u_v, gmax_v, ginv_v, starts_v, row_v, m_v, all_v,
             shared_max, shared_sum, sem_in, sem_out):
    sid = lax.axis_index("s")
    base = sid * _CHUNK

    cp_coh = pltpu.make_async_copy(coh_hbm.at[pl.ds(base, _CHUNK)], coh_v, sem_in)
    cp_idx = pltpu.make_async_copy(batch_hbm.at[pl.ds(base, _CHUNK)], idx_v, sem_in)
    cp_u = pltpu.make_async_copy(u_hbm, u_v, sem_in)
    cp_st = pltpu.make_async_copy(starts_hbm, starts_v, sem_in)
    cp_coh.start()
    cp_idx.start()
    cp_u.start()
    cp_st.start()
    cp_coh.wait()
    cp_idx.wait()
    cp_u.wait()
    cp_st.wait()

    lane = lax.iota(jnp.int32, _L)
    neg_inf_vec = jnp.full((_L,), _NEG_INF, dtype=jnp.float32)
    zero_vec = jnp.zeros((_L,), dtype=jnp.float32)
    inv_temp = jnp.float32(1.0 / _TEMPERATURE)

    def body_a(j):
        off = j * _L
        c = coh_v[pl.ds(off, _L)]
        ii = idx_v[pl.ds(off, _L)]
        ue = plsc.load_gather(u_v, [ii])
        s_v[pl.ds(off, _L)] = (c * ue) * inv_temp

    plsc.parallel_loop(0, _NVEC, unroll=4)(body_a)

    cp_prio = pltpu.make_async_copy(s_v.at[pl.ds(0, _CHUNK)], prio_hbm.at[pl.ds(base, _CHUNK)], sem_out)
    cp_prio.start()

    sv = starts_v[...]

    ranges = []
    for b in range(_B):
        lo_g = sv[b]
        hi_g = sv[b + 1] if b < _B - 1 else jnp.int32(_N)
        lo = jnp.clip(lo_g - base, 0, _CHUNK)
        hi = jnp.clip(hi_g - base, 0, _CHUNK)
        ranges.append((lo, hi))

    lane16 = lane * _L
    row_iotas = [lane + q * _L for q in range(_NVEC // _L)]

    def _range_reduce(src_ref, combine, reduce_fn, identity_vec):
        def pre_body(k, src_ref=src_ref, combine=combine,
                     identity_vec=identity_vec):
            macc = identity_vec
            for c in range(_L):
                g = plsc.load_gather(src_ref, [lane16 + (k * _L * _L + c)])
                macc = combine(macc, g)
            m_v[pl.ds(k * _L, _L)] = macc

        plsc.parallel_loop(0, _NVEC // _L, unroll=2)(pre_body)

        tab = identity_vec
        for b in range(_B):
            lo, hi = ranges[b]
            rlo = (lo + (_L - 1)) >> 4
            rhi = hi >> 4
            e_a = lo - (lo & (_L - 1))
            e_b = hi - (hi & (_L - 1))
            aend = jnp.minimum(hi, rlo * _L)
            bstart = jnp.maximum(aend, rhi * _L)

            accv = identity_vec
            for q in range(_NVEC // _L):
                rq = row_iotas[q]
                mq = jnp.logical_and(rq >= rlo, rq < rhi)
                accv = combine(
                    accv, jnp.where(mq, m_v[pl.ds(q * _L, _L)], identity_vec)
                )
            x_a = src_ref[pl.ds(e_a, _L)]
            pos_a = lane + e_a
            m_a = jnp.logical_and(pos_a >= lo, pos_a < aend)
            accv = combine(accv, jnp.where(m_a, x_a, identity_vec))
            x_b = src_ref[pl.ds(e_b, _L)]
            pos_b = lane + e_b
            m_b = jnp.logical_and(pos_b >= bstart, pos_b < hi)
            accv = combine(accv, jnp.where(m_b, x_b, identity_vec))
            tab = jnp.where(lane == b, combine(tab, reduce_fn(accv)), tab)
        return tab

    lmax = _range_reduce(s_v, jnp.maximum, jnp.max, neg_inf_vec)
    gmax_v[...] = lmax

    def body_b(j):
        off = j * _L
        s = s_v[pl.ds(off, _L)]
        ii = idx_v[pl.ds(off, _L)]
        lm = plsc.load_gather(gmax_v, [ii])
        e_v[pl.ds(off, _L)] = jnp.exp(s - lm)

    plsc.parallel_loop(0, _NVEC, unroll=4)(body_b)

    lsum = _range_reduce(e_v, jnp.add, jnp.sum, zero_vec)

    row_v[...] = lmax
    pltpu.sync_copy(row_v, shared_max.at[pl.ds(sid * _L, _L)])
    row_v[...] = lsum
    pltpu.sync_copy(row_v, shared_sum.at[pl.ds(sid * _L, _L)])
    plsc.subcore_barrier()

    pltpu.sync_copy(shared_max, all_v.at[pl.ds(0, _NTILES * _L)])
    pltpu.sync_copy(shared_sum, all_v.at[pl.ds(_NTILES * _L, _NTILES * _L)])
    g = neg_inf_vec
    for t in range(_NTILES):
        g = jnp.maximum(g, all_v[pl.ds(t * _L, _L)])
    total = zero_vec
    for t in range(_NTILES):
        lm_t = all_v[pl.ds(t * _L, _L)]
        ls_t = all_v[pl.ds(_NTILES * _L + t * _L, _L)]
        total = total + ls_t * jnp.exp(lm_t - g)
    ginv_v[...] = jnp.exp(lmax - g) / total

    def body_c(j):
        off = j * _L
        e = e_v[pl.ds(off, _L)]
        ii = idx_v[pl.ds(off, _L)]
        fv = plsc.load_gather(ginv_v, [ii])
        n_v[pl.ds(off, _L)] = e * fv

    plsc.parallel_loop(0, _NVEC, unroll=4)(body_c)

    pltpu.sync_copy(n_v, norm_hbm.at[pl.ds(base, _CHUNK)])
    cp_prio.wait()


def _sc_softmax(coherence_spatial, batch, uncertainty, starts):
    mesh = plsc.VectorSubcoreMesh(
        core_axis_name="c", subcore_axis_name="s", num_cores=1
    )
    f32 = jnp.float32
    run = functools.partial(
        pl.kernel,
        mesh=mesh,
        out_type=[
            jax.ShapeDtypeStruct((_N,), f32),
            jax.ShapeDtypeStruct((_N,), f32),
        ],
        scratch_types=[
            pltpu.VMEM((_CHUNK,), f32),
            pltpu.VMEM((_CHUNK,), jnp.int32),
            pltpu.VMEM((_CHUNK + _L,), f32),
            pltpu.VMEM((_CHUNK + _L,), f32),
            pltpu.VMEM((_CHUNK,), f32),
            pltpu.VMEM((_L,), f32),
            pltpu.VMEM((_L,), f32),
            pltpu.VMEM((_L,), f32),
            pltpu.VMEM((_L,), jnp.int32),
            pltpu.VMEM((_L,), f32),
            pltpu.VMEM((_NVEC,), f32),
            pltpu.VMEM((2 * _NTILES * _L,), f32),
            pltpu.VMEM_SHARED((_NTILES * _L,), f32),
            pltpu.VMEM_SHARED((_NTILES * _L,), f32),
            pltpu.SemaphoreType.DMA,
            pltpu.SemaphoreType.DMA,
        ],
        compiler_params=pltpu.CompilerParams(needs_layout_passes=False),
    )(_sc_body)
    return run(coherence_spatial, batch, uncertainty, starts)


def kernel(coherence_spatial, posterior_mean, posterior_std, batch):
    uncertainty, starts = _tc_prep(posterior_std, batch)
    priority, priority_normalized = _sc_softmax(
        coherence_spatial, batch, uncertainty, starts
    )
    return (priority, priority_normalized, uncertainty)


# speedup vs baseline: 1.0148x; 1.0148x over previous
import functools
import math

import jax
import jax.numpy as jnp
from jax import lax
from jax.experimental import pallas as pl
from jax.experimental.pallas import tpu as pltpu
from jax.experimental.pallas import tpu_sc as plsc

_B = 16
_N = 32768
_D = 1024
_TEMPERATURE = 1.0

_L = 16
_NTILES = 16
_CHUNK = _N // _NTILES
_NVEC = _CHUNK // _L

_NEG_INF = float("-inf")


def _tc_prep_body(std_ref, batch_ref, unc_ref, starts_ref):
    s = std_ref[...]
    ent = 0.5 * jnp.log((2.0 * math.pi * math.e) * jnp.square(s))
    unc_ref[...] = jnp.sum(ent, axis=1, keepdims=True)

    b2 = batch_ref[...]
    iota2 = lax.broadcasted_iota(jnp.int32, (_B, 1), 0)
    acc = jnp.zeros((_B, 1), jnp.int32)
    for b in range(_B):
        cnt = jnp.sum((b2 < b).astype(jnp.int32))
        acc = jnp.where(iota2 == b, cnt, acc)
    starts_ref[...] = acc


def _tc_prep(posterior_std, batch):
    unc, starts = pl.pallas_call(
        _tc_prep_body,
        out_shape=[
            jax.ShapeDtypeStruct((_B, 1), jnp.float32),
            jax.ShapeDtypeStruct((_B, 1), jnp.int32),
        ],
    )(posterior_std, batch.reshape(_B * _L, -1))
    return unc.reshape(_B), starts.reshape(_B)


def _sc_body(coh_hbm, batch_hbm, u_hbm, starts_hbm, prio_hbm, norm_hbm,
             coh_v, idx_v, s_v, e_v, n_v,
             u_v, gmax_v, ginv_v, starts_v, row_v, all_v,
             shared_max, shared_sum, sem_in, sem_out):
    sid = lax.axis_index("s")
    base = sid * _CHUNK

    cp_coh = pltpu.make_async_copy(coh_hbm.at[pl.ds(base, _CHUNK)], coh_v, sem_in)
    cp_idx = pltpu.make_async_copy(batch_hbm.at[pl.ds(base, _CHUNK)], idx_v, sem_in)
    cp_u = pltpu.make_async_copy(u_hbm, u_v, sem_in)
    cp_st = pltpu.make_async_copy(starts_hbm, starts_v, sem_in)
    cp_coh.start()
    cp_idx.start()
    cp_u.start()
    cp_st.start()
    cp_coh.wait()
    cp_idx.wait()
    cp_u.wait()
    cp_st.wait()

    lane = lax.iota(jnp.int32, _L)
    neg_inf_vec = jnp.full((_L,), _NEG_INF, dtype=jnp.float32)
    zero_vec = jnp.zeros((_L,), dtype=jnp.float32)
    inv_temp = jnp.float32(1.0 / _TEMPERATURE)

    def body_a(j):
        off = j * _L
        c = coh_v[pl.ds(off, _L)]
        ii = idx_v[pl.ds(off, _L)]
        ue = plsc.load_gather(u_v, [ii])
        s_v[pl.ds(off, _L)] = (c * ue) * inv_temp

    plsc.parallel_loop(0, _NVEC, unroll=8)(body_a)

    cp_prio = pltpu.make_async_copy(s_v, prio_hbm.at[pl.ds(base, _CHUNK)], sem_out)
    cp_prio.start()

    sv = starts_v[...]

    ranges = []
    for b in range(_B):
        lo_g = sv[b]
        hi_g = sv[b + 1] if b < _B - 1 else jnp.int32(_N)
        lo = jnp.clip(lo_g - base, 0, _CHUNK)
        hi = jnp.clip(hi_g - base, 0, _CHUNK)
        ranges.append((lo, hi))

    def _range_reduce(src_ref, combine, reduce_fn, identity_vec):
        tab = identity_vec
        for b in range(_B):
            lo, hi = ranges[b]
            jlo = lo >> 4
            jhi = (hi + (_L - 1)) >> 4

            def red_body(j, acc, lo=lo, hi=hi, src_ref=src_ref,
                         combine=combine, identity_vec=identity_vec):
                pos = lane + j * _L
                x = src_ref[pl.ds(j * _L, _L)]
                m = jnp.logical_and(pos >= lo, pos < hi)
                return combine(acc, jnp.where(m, x, identity_vec))

            acc = lax.fori_loop(jlo, jhi, red_body, identity_vec)
            tab = jnp.where(lane == b, combine(tab, reduce_fn(acc)), tab)
        return tab

    lmax = _range_reduce(s_v, jnp.maximum, jnp.max, neg_inf_vec)
    gmax_v[...] = lmax

    def body_b(j):
        off = j * _L
        s = s_v[pl.ds(off, _L)]
        ii = idx_v[pl.ds(off, _L)]
        lm = plsc.load_gather(gmax_v, [ii])
        e_v[pl.ds(off, _L)] = jnp.exp(s - lm)

    plsc.parallel_loop(0, _NVEC, unroll=8)(body_b)

    lsum = _range_reduce(e_v, jnp.add, jnp.sum, zero_vec)

    row_v[...] = lmax
    pltpu.sync_copy(row_v, shared_max.at[pl.ds(sid * _L, _L)])
    row_v[...] = lsum
    pltpu.sync_copy(row_v, shared_sum.at[pl.ds(sid * _L, _L)])
    plsc.subcore_barrier()

    pltpu.sync_copy(shared_max, all_v.at[pl.ds(0, _NTILES * _L)])
    pltpu.sync_copy(shared_sum, all_v.at[pl.ds(_NTILES * _L, _NTILES * _L)])
    g = neg_inf_vec
    for t in range(_NTILES):
        g = jnp.maximum(g, all_v[pl.ds(t * _L, _L)])
    total = zero_vec
    for t in range(_NTILES):
        lm_t = all_v[pl.ds(t * _L, _L)]
        ls_t = all_v[pl.ds(_NTILES * _L + t * _L, _L)]
        total = total + ls_t * jnp.exp(lm_t - g)
    ginv_v[...] = jnp.exp(lmax - g) / total

    def body_c(j):
        off = j * _L
        e = e_v[pl.ds(off, _L)]
        ii = idx_v[pl.ds(off, _L)]
        fv = plsc.load_gather(ginv_v, [ii])
        n_v[pl.ds(off, _L)] = e * fv

    plsc.parallel_loop(0, _NVEC, unroll=8)(body_c)

    pltpu.sync_copy(n_v, norm_hbm.at[pl.ds(base, _CHUNK)])
    cp_prio.wait()


def _sc_softmax(coherence_spatial, batch, uncertainty, starts):
    mesh = plsc.VectorSubcoreMesh(
        core_axis_name="c", subcore_axis_name="s", num_cores=1
    )
    f32 = jnp.float32
    run = functools.partial(
        pl.kernel,
        mesh=mesh,
        out_type=[
            jax.ShapeDtypeStruct((_N,), f32),
            jax.ShapeDtypeStruct((_N,), f32),
        ],
        scratch_types=[
            pltpu.VMEM((_CHUNK,), f32),
            pltpu.VMEM((_CHUNK,), jnp.int32),
            pltpu.VMEM((_CHUNK,), f32),
            pltpu.VMEM((_CHUNK,), f32),
            pltpu.VMEM((_CHUNK,), f32),
            pltpu.VMEM((_L,), f32),
            pltpu.VMEM((_L,), f32),
            pltpu.VMEM((_L,), f32),
            pltpu.VMEM((_L,), jnp.int32),
            pltpu.VMEM((_L,), f32),
            pltpu.VMEM((2 * _NTILES * _L,), f32),
            pltpu.VMEM_SHARED((_NTILES * _L,), f32),
            pltpu.VMEM_SHARED((_NTILES * _L,), f32),
            pltpu.SemaphoreType.DMA,
            pltpu.SemaphoreType.DMA,
        ],
        compiler_params=pltpu.CompilerParams(needs_layout_passes=False),
    )(_sc_body)
    return run(coherence_spatial, batch, uncertainty, starts)


def kernel(coherence_spatial, posterior_mean, posterior_std, batch):
    uncertainty, starts = _tc_prep(posterior_std, batch)
    priority, priority_normalized = _sc_softmax(
        coherence_spatial, batch, uncertainty, starts
    )
    return (priority, priority_normalized, uncertainty)


# speedup vs baseline: 1.0246x; 1.0097x over previous
import functools
import math

import jax
import jax.numpy as jnp
from jax import lax
from jax.experimental import pallas as pl
from jax.experimental.pallas import tpu as pltpu
from jax.experimental.pallas import tpu_sc as plsc

_B = 16
_N = 32768
_D = 1024
_TEMPERATURE = 1.0

_L = 16
_NTILES = 16
_CHUNK = _N // _NTILES
_NVEC = _CHUNK // _L

_NEG_INF = float("-inf")


def _tc_prep_body(std_ref, batch_ref, unc_ref, starts_ref):
    s = std_ref[...]
    ent = 0.5 * jnp.log((2.0 * math.pi * math.e) * jnp.square(s))
    unc_ref[...] = jnp.sum(ent, axis=1, keepdims=True)

    b2 = batch_ref[...]
    iota2 = lax.broadcasted_iota(jnp.int32, (_B, 1), 0)
    acc = jnp.zeros((_B, 1), jnp.int32)
    for b in range(_B):
        cnt = jnp.sum((b2 < b).astype(jnp.int32))
        acc = jnp.where(iota2 == b, cnt, acc)
    starts_ref[...] = acc


def _tc_prep(posterior_std, batch):
    unc, starts = pl.pallas_call(
        _tc_prep_body,
        out_shape=[
            jax.ShapeDtypeStruct((_B, 1), jnp.float32),
            jax.ShapeDtypeStruct((_B, 1), jnp.int32),
        ],
    )(posterior_std, batch.reshape(_B * _L, -1))
    return unc.reshape(_B), starts.reshape(_B)


def _sc_body(coh_hbm, batch_hbm, u_hbm, starts_hbm, prio_hbm, norm_hbm,
             coh_v, idx_v, s_v, e_v, n_v,
             u_v, gmax_v, ginv_v, starts_v, row_v, all_v,
             shared_max, shared_sum, sem_in, sem_out):
    sid = lax.axis_index("s")
    base = sid * _CHUNK

    cp_coh = pltpu.make_async_copy(coh_hbm.at[pl.ds(base, _CHUNK)], coh_v, sem_in)
    cp_idx = pltpu.make_async_copy(batch_hbm.at[pl.ds(base, _CHUNK)], idx_v, sem_in)
    cp_u = pltpu.make_async_copy(u_hbm, u_v, sem_in)
    cp_st = pltpu.make_async_copy(starts_hbm, starts_v, sem_in)
    cp_coh.start()
    cp_idx.start()
    cp_u.start()
    cp_st.start()
    cp_coh.wait()
    cp_idx.wait()
    cp_u.wait()
    cp_st.wait()

    lane = lax.iota(jnp.int32, _L)
    neg_inf_vec = jnp.full((_L,), _NEG_INF, dtype=jnp.float32)
    zero_vec = jnp.zeros((_L,), dtype=jnp.float32)
    inv_temp = jnp.float32(1.0 / _TEMPERATURE)

    def body_a(j):
        off = j * _L
        c = coh_v[pl.ds(off, _L)]
        ii = idx_v[pl.ds(off, _L)]
        ue = plsc.load_gather(u_v, [ii])
        s_v[pl.ds(off, _L)] = (c * ue) * inv_temp

    plsc.parallel_loop(0, _NVEC, unroll=4)(body_a)

    cp_prio = pltpu.make_async_copy(s_v, prio_hbm.at[pl.ds(base, _CHUNK)], sem_out)
    cp_prio.start()

    sv = starts_v[...]

    ranges = []
    for b in range(_B):
        lo_g = sv[b]
        hi_g = sv[b + 1] if b < _B - 1 else jnp.int32(_N)
        lo = jnp.clip(lo_g - base, 0, _CHUNK)
        hi = jnp.clip(hi_g - base, 0, _CHUNK)
        ranges.append((lo, hi))

    def _range_reduce(src_ref, combine, reduce_fn, identity_vec):
        tab = identity_vec
        for b in range(_B):
            lo, hi = ranges[b]
            jlo = lo >> 4
            jhi = (hi + (_L - 1)) >> 4

            def red_body(j, acc, lo=lo, hi=hi, src_ref=src_ref,
                         combine=combine, identity_vec=identity_vec):
                pos = lane + j * _L
                x = src_ref[pl.ds(j * _L, _L)]
                m = jnp.logical_and(pos >= lo, pos < hi)
                return combine(acc, jnp.where(m, x, identity_vec))

            acc = lax.fori_loop(jlo, jhi, red_body, identity_vec)
            tab = jnp.where(lane == b, combine(tab, reduce_fn(acc)), tab)
        return tab

    lmax = _range_reduce(s_v, jnp.maximum, jnp.max, neg_inf_vec)
    gmax_v[...] = lmax

    def body_b(j):
        off = j * _L
        s = s_v[pl.ds(off, _L)]
        ii = idx_v[pl.ds(off, _L)]
        lm = plsc.load_gather(gmax_v, [ii])
        e_v[pl.ds(off, _L)] = jnp.exp(s - lm)

    plsc.parallel_loop(0, _NVEC, unroll=4)(body_b)

    lsum = _range_reduce(e_v, jnp.add, jnp.sum, zero_vec)

    row_v[...] = lmax
    pltpu.sync_copy(row_v, shared_max.at[pl.ds(sid * _L, _L)])
    row_v[...] = lsum
    pltpu.sync_copy(row_v, shared_sum.at[pl.ds(sid * _L, _L)])
    plsc.subcore_barrier()

    pltpu.sync_copy(shared_max, all_v.at[pl.ds(0, _NTILES * _L)])
    pltpu.sync_copy(shared_sum, all_v.at[pl.ds(_NTILES * _L, _NTILES * _L)])
    g = neg_inf_vec
    for t in range(_NTILES):
        g = jnp.maximum(g, all_v[pl.ds(t * _L, _L)])
    total = zero_vec
    for t in range(_NTILES):
        lm_t = all_v[pl.ds(t * _L, _L)]
        ls_t = all_v[pl.ds(_NTILES * _L + t * _L, _L)]
        total = total + ls_t * jnp.exp(lm_t - g)
    ginv_v[...] = jnp.exp(lmax - g) / total

    def body_c(j):
        off = j * _L
        e = e_v[pl.ds(off, _L)]
        ii = idx_v[pl.ds(off, _L)]
        fv = plsc.load_gather(ginv_v, [ii])
        n_v[pl.ds(off, _L)] = e * fv

    plsc.parallel_loop(0, _NVEC, unroll=4)(body_c)

    pltpu.sync_copy(n_v, norm_hbm.at[pl.ds(base, _CHUNK)])
    cp_prio.wait()


def _sc_softmax(coherence_spatial, batch, uncertainty, starts):
    mesh = plsc.VectorSubcoreMesh(
        core_axis_name="c", subcore_axis_name="s", num_cores=1
    )
    f32 = jnp.float32
    run = functools.partial(
        pl.kernel,
        mesh=mesh,
        out_type=[
            jax.ShapeDtypeStruct((_N,), f32),
            jax.ShapeDtypeStruct((_N,), f32),
        ],
        scratch_types=[
            pltpu.VMEM((_CHUNK,), f32),
            pltpu.VMEM((_CHUNK,), jnp.int32),
            pltpu.VMEM((_CHUNK,), f32),
            pltpu.VMEM((_CHUNK,), f32),
            pltpu.VMEM((_CHUNK,), f32),
            pltpu.VMEM((_L,), f32),
            pltpu.VMEM((_L,), f32),
            pltpu.VMEM((_L,), f32),
            pltpu.VMEM((_L,), jnp.int32),
            pltpu.VMEM((_L,), f32),
            pltpu.VMEM((2 * _NTILES * _L,), f32),
            pltpu.VMEM_SHARED((_NTILES * _L,), f32),
            pltpu.VMEM_SHARED((_NTILES * _L,), f32),
            pltpu.SemaphoreType.DMA,
            pltpu.SemaphoreType.DMA,
        ],
        compiler_params=pltpu.CompilerParams(needs_layout_passes=False),
    )(_sc_body)
    return run(coherence_spatial, batch, uncertainty, starts)


def kernel(coherence_spatial, posterior_mean, posterior_std, batch):
    uncertainty, starts = _tc_prep(posterior_std, batch)
    priority, priority_normalized = _sc_softmax(
        coherence_spatial, batch, uncertainty, starts
    )
    return (priority, priority_normalized, uncertainty)
